# baseline (speedup 1.0000x reference)
import functools

import jax
import jax.numpy as jnp
from jax import lax
from jax.experimental import pallas as pl
from jax.experimental.pallas import tpu as pltpu
from jax.experimental.pallas import tpu_sc as plsc

NC = 2
NS = 16
NW = NC * NS
LANES = 16
NB = 5


def _make_deg(n, e):
    epw = e // NW
    mesh = plsc.VectorSubcoreMesh(core_axis_name="c", subcore_axis_name="s")

    @functools.partial(
        pl.kernel,
        out_type=jax.ShapeDtypeStruct((NW, n), jnp.float32),
        mesh=mesh,
        compiler_params=pltpu.CompilerParams(needs_layout_passes=False),
        scratch_types=[
            pltpu.VMEM((epw,), jnp.int32),
            pltpu.VMEM((n,), jnp.float32),
        ],
    )
    def deg_kernel(dstw_hbm, out_hbm, dst_v, hist_v):
        c = lax.axis_index("c")
        s = lax.axis_index("s")
        w = c * NS + s
        pltpu.sync_copy(dstw_hbm.at[w], dst_v)

        def zero(i, carry):
            hist_v[pl.ds(i * LANES, LANES)] = jnp.zeros((LANES,), jnp.float32)
            return carry

        lax.fori_loop(0, n // LANES, zero, None)

        ones = jnp.ones((LANES,), jnp.float32)

        def hist(i, carry):
            idx = dst_v[pl.ds(i * LANES, LANES)]
            plsc.addupdate_scatter(hist_v, [idx], ones)
            return carry

        lax.fori_loop(0, epw // LANES, hist, None)
        pltpu.sync_copy(hist_v, out_hbm.at[w])

    return deg_kernel


def _make_scatfin(n, e, dh, ch):
    eps = e // NS
    nch = eps // ch
    rpt = n // NS
    fch = rpt // 5
    dwin = n - (NS - 1) * (rpt - 1) + LANES
    wlo = rpt - 1
    mesh = plsc.VectorSubcoreMesh(core_axis_name="c", subcore_axis_name="s")

    @functools.partial(
        pl.kernel,
        out_type=jax.ShapeDtypeStruct((NC, n, dh), jnp.float32),
        mesh=mesh,
        compiler_params=pltpu.CompilerParams(use_tc_tiling_on_sc=False),
        scratch_types=[
            pltpu.VMEM((nch, ch), jnp.int32),
            pltpu.VMEM((nch, ch), jnp.int32),
            pltpu.VMEM((NB, ch, dh), jnp.float32),
            pltpu.VMEM((rpt // 5, dh), jnp.float32),
            pltpu.VMEM((rpt // 5, dh), jnp.float32),
            pltpu.VMEM((n - (NS - 1) * (n // NS - 1) + LANES,), jnp.float32),
            pltpu.VMEM((dh,), jnp.float32),
            pltpu.VMEM_SHARED((n, dh), jnp.float32),
            pltpu.SemaphoreType.DMA,
            pltpu.SemaphoreType.DMA,
            pltpu.SemaphoreType.DMA,
            pltpu.SemaphoreType.DMA,
            pltpu.SemaphoreType.DMA,
        ],
    )
    def scatfin_kernel(ys_hbm, srcr_hbm, dstr_hbm, dis_hbm, b2_hbm, zrows_hbm,
                       out_hbm, src_v, dst_v, buf_v, acc_c, y_c, dis_v, b_v,
                       acc_sh, sem0, sem1, sem2, sem3, sem4):
        sems = (sem0, sem1, sem2, sem3, sem4)
        c = lax.axis_index("c")
        s = lax.axis_index("s")
        pltpu.sync_copy(srcr_hbm.at[s], src_v)
        pltpu.sync_copy(dstr_hbm.at[s], dst_v)
        pltpu.sync_copy(dis_hbm.at[pl.ds(s * wlo, dwin)], dis_v)
        pltpu.sync_copy(b2_hbm.at[c], b_v)
        pltpu.sync_copy(zrows_hbm, acc_sh.at[pl.ds(s * rpt, rpt)])

        cn = c * n

        def shift(i, carry):
            q = i // (ch // LANES)
            k = i % (ch // LANES)
            sl = (q, pl.ds(k * LANES, LANES))
            src_v[sl] = src_v[sl] + cn
            return carry

        lax.fori_loop(0, nch * (ch // LANES), shift, None)
        plsc.subcore_barrier()

        for b in range(NB):
            pltpu.async_copy(ys_hbm.at[src_v.at[b]], buf_v.at[b], sems[b])

        def group(g, carry):
            for b in range(NB):
                j = g * NB + b
                pltpu.make_async_copy(
                    ys_hbm.at[pl.ds(0, ch)], buf_v.at[b], sems[b]).wait()
                pltpu.sync_copy(buf_v.at[b], acc_sh.at[dst_v.at[j]], add=True)
                jn = j + NB

                @pl.when(jn < nch)
                def _():
                    pltpu.async_copy(
                        ys_hbm.at[src_v.at[jn]], buf_v.at[b], sems[b])
            return carry

        lax.fori_loop(0, nch // NB, group, None)
        plsc.subcore_barrier()

        b_regs = [b_v[pl.ds(k * LANES, LANES)] for k in range(dh // LANES)]

        def fin_chunk(q, carry):
            base = s * rpt + q * fch
            pltpu.sync_copy(acc_sh.at[pl.ds(base, fch)], acc_c)
            pltpu.sync_copy(ys_hbm.at[pl.ds(cn + base, fch)], y_c)

            def fin_row(i, carry2):
                dval = dis_v[pl.ds(s + q * fch + i, LANES)][0]
                for k in range(dh // LANES):
                    sl = (i, pl.ds(k * LANES, LANES))
                    acc_c[sl] = (acc_c[sl] + y_c[sl]) * dval + b_regs[k]
                return carry2

            lax.fori_loop(0, fch, fin_row, None)
            pltpu.sync_copy(acc_c, out_hbm.at[c, pl.ds(base, fch)])
            return carry

        lax.fori_loop(0, rpt // fch, fin_chunk, None)

    return scatfin_kernel


def _mm_body(x_ref, w_ref, dp_ref, ys_ref, dis_ref):
    n = x_ref.shape[0]
    dh = w_ref.shape[1] // 2
    xw = jnp.dot(x_ref[...], w_ref[...], preferred_element_type=jnp.float32)
    deg = jnp.sum(dp_ref[...], axis=0) + 1.0
    dis = lax.rsqrt(deg)
    y = xw * dis[:, None]
    ys_ref[0:n, :] = y[:, :dh]
    ys_ref[n:2 * n, :] = y[:, dh:]
    dis_ref[...] = dis


def kernel(x, edge_index, W, b):
    n, d_in = x.shape
    d_out = W.shape[1]
    dh = d_out // 2
    e = edge_index.shape[1]
    ch = 80
    epw = e // NW
    eps = e // NS

    src_r = edge_index[0].reshape(NS, eps // ch, ch)
    dst_r = edge_index[1].reshape(NS, eps // ch, ch)
    dst_w = edge_index[1].reshape(NW, epw)

    deg_parts = _make_deg(n, e)(dst_w)

    y_stack, dis = pl.pallas_call(
        _mm_body,
        out_shape=[
            jax.ShapeDtypeStruct((2 * n, dh), jnp.float32),
            jax.ShapeDtypeStruct((n,), jnp.float32),
        ],
    )(x, W, deg_parts)

    zrows = jnp.zeros((n // NS, dh), jnp.float32)
    b2 = b.reshape(NC, dh)
    dis_pad = jnp.concatenate([dis, jnp.zeros((LANES,), jnp.float32)])
    out2 = _make_scatfin(n, e, dh, ch)(
        y_stack, src_r, dst_r, dis_pad, b2, zrows)
    return jnp.concatenate([out2[0], out2[1]], axis=1)


# speedup vs baseline: 1.0413x; 1.0413x over previous
import functools

import jax
import jax.numpy as jnp
from jax import lax
from jax.experimental import pallas as pl
from jax.experimental.pallas import tpu as pltpu
from jax.experimental.pallas import tpu_sc as plsc

NC = 2
NS = 16
LANES = 16
NB = 5


def _make_deg(n, e):
    epw = e // (NC * NS)
    rpt = n // NS
    wlo = rpt - 1
    dwin = n - (NS - 1) * wlo + LANES
    npad = n + LANES
    mesh = plsc.VectorSubcoreMesh(core_axis_name="c", subcore_axis_name="s")

    @functools.partial(
        pl.kernel,
        out_type=jax.ShapeDtypeStruct((NC, NS, dwin), jnp.float32),
        mesh=mesh,
        compiler_params=pltpu.CompilerParams(
            needs_layout_passes=False, use_tc_tiling_on_sc=False),
        scratch_types=[
            pltpu.VMEM((epw,), jnp.int32),
            pltpu.VMEM((n,), jnp.float32),
            pltpu.VMEM((dwin,), jnp.float32),
            pltpu.VMEM((dwin,), jnp.float32),
            pltpu.VMEM_SHARED((NS, npad), jnp.float32),
        ],
    )
    def deg_kernel(dstw_hbm, out_hbm, dst_v, hist_v, wsum_v, wtmp_v, deg_sh):
        c = lax.axis_index("c")
        s = lax.axis_index("s")
        w = c * NS + s
        pltpu.sync_copy(dstw_hbm.at[w], dst_v)

        def zero(i, carry):
            hist_v[pl.ds(i * LANES, LANES)] = jnp.zeros((LANES,), jnp.float32)
            return carry

        lax.fori_loop(0, n // LANES, zero, None)

        ones = jnp.ones((LANES,), jnp.float32)

        def hist(i, carry):
            idx = dst_v[pl.ds(i * LANES, LANES)]
            plsc.addupdate_scatter(hist_v, [idx], ones)
            return carry

        lax.fori_loop(0, epw // LANES, hist, None)
        pltpu.sync_copy(hist_v, deg_sh.at[s, pl.ds(0, n)])
        plsc.subcore_barrier()
        pltpu.sync_copy(deg_sh.at[0, pl.ds(s * wlo, dwin)], wsum_v)

        def accw(t, carry):
            pltpu.sync_copy(deg_sh.at[t, pl.ds(s * wlo, dwin)], wtmp_v)

            def addw(i, carry2):
                sl = pl.ds(i * LANES, LANES)
                wsum_v[sl] = wsum_v[sl] + wtmp_v[sl]
                return carry2

            lax.fori_loop(0, dwin // LANES, addw, None)
            return carry

        lax.fori_loop(1, NS, accw, None)
        pltpu.sync_copy(wsum_v, out_hbm.at[c, s])

    return deg_kernel


def _make_sc(n, e, dh, ch):
    eps = e // NS
    nch = eps // ch
    rpt = n // NS
    fch = rpt // 5
    wlo = rpt - 1
    dwin = n - (NS - 1) * wlo + LANES
    nw = NC * NS
    mesh = plsc.VectorSubcoreMesh(core_axis_name="c", subcore_axis_name="s")

    @functools.partial(
        pl.kernel,
        out_type=[
            jax.ShapeDtypeStruct((n, dh), jnp.float32),
            jax.ShapeDtypeStruct((n, dh), jnp.float32),
        ],
        mesh=mesh,
        compiler_params=pltpu.CompilerParams(
            use_tc_tiling_on_sc=False, needs_layout_passes=False),
        scratch_types=[
            pltpu.VMEM((nch, ch), jnp.int32),
            pltpu.VMEM((nch, ch), jnp.int32),
            pltpu.VMEM((NB, ch, dh), jnp.float32),
            pltpu.VMEM((fch, dh), jnp.float32),
            pltpu.VMEM((fch, dh), jnp.float32),
            pltpu.VMEM((NC, dwin), jnp.float32),
            pltpu.VMEM((dwin,), jnp.float32),
            pltpu.VMEM((dh,), jnp.float32),
            pltpu.VMEM_SHARED((n, dh), jnp.float32),
            pltpu.SemaphoreType.DMA,
            pltpu.SemaphoreType.DMA,
            pltpu.SemaphoreType.DMA,
            pltpu.SemaphoreType.DMA,
            pltpu.SemaphoreType.DMA,
        ],
    )
    def sc_kernel(xws_hbm, srcr_hbm, dstr_hbm, degp_hbm, b2_hbm, zrows_hbm,
                  o0_hbm, o1_hbm, src_v, dst_v, buf_v, acc_c, y_c, dwin_v,
                  dis_v, b_v, acc_sh, sem0, sem1, sem2, sem3, sem4):
        sems = (sem0, sem1, sem2, sem3, sem4)
        c = lax.axis_index("c")
        s = lax.axis_index("s")
        pltpu.sync_copy(srcr_hbm.at[s], src_v)
        pltpu.sync_copy(dstr_hbm.at[s], dst_v)
        pltpu.sync_copy(b2_hbm.at[c], b_v)
        pltpu.sync_copy(zrows_hbm, acc_sh.at[pl.ds(s * rpt, rpt)])

        pltpu.sync_copy(degp_hbm.at[0, s], dwin_v.at[0])
        pltpu.sync_copy(degp_hbm.at[1, s], dwin_v.at[1])

        def wsum(i, carry):
            sl = pl.ds(i * LANES, LANES)
            dis_v[sl] = dwin_v[0, sl] + dwin_v[1, sl]
            return carry

        lax.fori_loop(0, dwin // LANES, wsum, None)

        def newton(i, carry):
            sl = pl.ds(i * LANES, LANES)
            d = dis_v[sl] + 1.0
            bits = plsc.bitcast(d, jnp.int32)
            g = plsc.bitcast(
                jnp.int32(0x5F3759DF) - (bits >> 1), jnp.float32)
            hd = d * 0.5
            for _ in range(3):
                g = g * (1.5 - hd * g * g)
            dis_v[sl] = g
            return carry

        lax.fori_loop(0, dwin // LANES, newton, None)

        b_regs = [b_v[pl.ds(k * LANES, LANES)] for k in range(dh // LANES)]

        def run_phases(y_ref, cn):
            def scale_chunk(q, carry):
                base = s * rpt + q * fch
                pltpu.sync_copy(xws_hbm.at[pl.ds(cn + base, fch)], y_c)

                def scale_row(i, carry2):
                    dval = dis_v[pl.ds(s + q * fch + i, LANES)][0]
                    for k in range(dh // LANES):
                        sl = (i, pl.ds(k * LANES, LANES))
                        y_c[sl] = y_c[sl] * dval
                    return carry2

                lax.fori_loop(0, fch, scale_row, None)
                pltpu.sync_copy(y_c, y_ref.at[pl.ds(base, fch)])
                return carry

            lax.fori_loop(0, rpt // fch, scale_chunk, None)
            plsc.subcore_barrier()

            for b in range(NB):
                pltpu.async_copy(y_ref.at[src_v.at[b]], buf_v.at[b], sems[b])

            def group(g, carry):
                for b in range(NB):
                    j = g * NB + b
                    pltpu.make_async_copy(
                        y_ref.at[pl.ds(0, ch)], buf_v.at[b], sems[b]).wait()
                    pltpu.sync_copy(
                        buf_v.at[b], acc_sh.at[dst_v.at[j]], add=True)
                    jn = j + NB

                    @pl.when(jn < nch)
                    def _():
                        pltpu.async_copy(
                            y_ref.at[src_v.at[jn]], buf_v.at[b], sems[b])
                return carry

            lax.fori_loop(0, nch // NB, group, None)
            plsc.subcore_barrier()

            def fin_chunk(q, carry):
                base = s * rpt + q * fch
                pltpu.sync_copy(acc_sh.at[pl.ds(base, fch)], acc_c)
                pltpu.sync_copy(y_ref.at[pl.ds(base, fch)], y_c)

                def fin_row(i, carry2):
                    dval = dis_v[pl.ds(s + q * fch + i, LANES)][0]
                    for k in range(dh // LANES):
                        sl = (i, pl.ds(k * LANES, LANES))
                        acc_c[sl] = (acc_c[sl] + y_c[sl]) * dval + b_regs[k]
                    return carry2

                lax.fori_loop(0, fch, fin_row, None)
                pltpu.sync_copy(acc_c, y_ref.at[pl.ds(base, fch)])
                return carry

            lax.fori_loop(0, rpt // fch, fin_chunk, None)

        @pl.when(c == 0)
        def _():
            run_phases(o0_hbm, 0)

        @pl.when(c == 1)
        def _():
            run_phases(o1_hbm, n)

    return sc_kernel


def _mm_body(x_ref, w_ref, ys_ref):
    n = x_ref.shape[0]
    dh = w_ref.shape[1] // 2
    xw = jnp.dot(x_ref[...], w_ref[...], preferred_element_type=jnp.float32)
    ys_ref[0:n, :] = xw[:, :dh]
    ys_ref[n:2 * n, :] = xw[:, dh:]


def kernel(x, edge_index, W, b):
    n, d_in = x.shape
    d_out = W.shape[1]
    dh = d_out // 2
    e = edge_index.shape[1]
    ch = 80
    eps = e // NS

    src_r = edge_index[0].reshape(NS, eps // ch, ch)
    dst_r = edge_index[1].reshape(NS, eps // ch, ch)
    dst_w = edge_index[1].reshape(NC * NS, e // (NC * NS))

    deg_parts = _make_deg(n, e)(dst_w)
    xw_stack = pl.pallas_call(
        _mm_body,
        out_shape=jax.ShapeDtypeStruct((2 * n, dh), jnp.float32),
    )(x, W)

    zrows = jnp.zeros((n // NS, dh), jnp.float32)
    b2 = b.reshape(NC, dh)
    o0, o1 = _make_sc(n, e, dh, ch)(
        xw_stack, src_r, dst_r, deg_parts, b2, zrows)
    return jnp.concatenate([o0, o1], axis=1)
